# baseline (speedup 1.0000x reference)
import functools

import jax
import jax.numpy as jnp
from jax import lax
from jax.experimental import pallas as pl
from jax.experimental.pallas import tpu as pltpu
from jax.experimental.pallas import tpu_sc as plsc

_N = 4096
_N_DEPTH = 3
_N_ORI = 100
_N_GEN = 10
_N_BINS = 50
_NPLANE = _N_DEPTH * _N_GEN
_NW = 32
_DOCS_PER_W = _N // _NW
_NQ = _DOCS_PER_W // 16
_TAB = 176


_HSTRIDE = _TAB
_HWORDS = _DOCS_PER_W * _HSTRIDE


def _sc_body(planes_h, wtab_h, params_h, out0_h, out1_h,
             wtab, params, buf0, buf1, hist, row0, row1, sem0, sem1):
    c = lax.axis_index("c")
    s = lax.axis_index("s")
    wid = s * 2 + c
    col = wid * _DOCS_PER_W

    pltpu.sync_copy(wtab_h, wtab)
    pltpu.sync_copy(params_h, params)

    bufs = (buf0, buf1)
    sems = (sem0, sem1)
    handles = [None, None]
    handles[0] = pltpu.async_copy(
        planes_h.at[0, 0, slice(None), pl.ds(col, _DOCS_PER_W)],
        bufs[0], sems[0])

    zeros16 = jnp.zeros((16,), jnp.float32)

    @plsc.parallel_loop(0, _HWORDS // 16, unroll=4)
    def _zero(i):
        hist[pl.ds(i * 16, 16)] = zeros16

    lanes = lax.iota(jnp.int32, 16)
    ones16 = jnp.ones((16,), jnp.float32)

    for p in range(_NPLANE):
        b = p & 1
        handles[b].wait()
        if p + 1 < _NPLANE:
            d1, g1 = divmod(p + 1, _N_GEN)
            handles[1 - b] = pltpu.async_copy(
                planes_h.at[d1, g1, slice(None), pl.ds(col, _DOCS_PER_W)],
                bufs[1 - b], sems[1 - b])

        buf = bufs[b]
        off = (p // _N_GEN) * _N_BINS
        dq = [lanes * _HSTRIDE + (16 * q * _HSTRIDE + off) for q in range(_NQ)]

        @plsc.parallel_loop(0, _N_ORI, unroll=2)
        def body(r, _buf=buf, _dq=dq):
            for q in range(_NQ):
                x = _buf[r, pl.ds(16 * q, 16)]
                t = jnp.minimum(x * jnp.float32(_N_BINS),
                                jnp.float32(_N_BINS - 1))
                idx = t.astype(jnp.int32) + _dq[q]
                plsc.addupdate_scatter(hist, [idx], ones16)

    dqb = [lanes * _HSTRIDE + 16 * q * _HSTRIDE for q in range(_NQ)]
    sums0 = tuple(jnp.zeros((16,), jnp.float32) for _ in range(_NQ))

    def dot_step(k, acc_t):
        ws = wtab[pl.ds(k, 16)][0]
        new = []
        for q in range(_NQ):
            h = plsc.load_gather(hist, [dqb[q] + k])
            new.append(acc_t[q] + h * ws)
        return tuple(new)

    sums = lax.fori_loop(0, _N_DEPTH * _N_BINS, dot_step, sums0)

    a0 = params[pl.ds(0, 16)]
    a1 = params[pl.ds(16, 16)]
    c0 = params[pl.ds(32, 16)]
    c1 = params[pl.ds(48, 16)]
    for q in range(_NQ):
        row0[pl.ds(q * 16, 16)] = sums[q] * a0 + c0
        row1[pl.ds(q * 16, 16)] = sums[q] * a1 + c1

    pltpu.sync_copy(row0, out0_h.at[pl.ds(col, _DOCS_PER_W)])
    pltpu.sync_copy(row1, out1_h.at[pl.ds(col, _DOCS_PER_W)])


_mesh = plsc.VectorSubcoreMesh(core_axis_name="c", subcore_axis_name="s")

_sc_call = functools.partial(
    pl.kernel,
    mesh=_mesh,
    compiler_params=pltpu.CompilerParams(needs_layout_passes=False),
    out_type=[
        jax.ShapeDtypeStruct((_N,), jnp.float32),
        jax.ShapeDtypeStruct((_N,), jnp.float32),
    ],
    scratch_types=[
        pltpu.VMEM((_TAB,), jnp.float32),
        pltpu.VMEM((64,), jnp.float32),
        pltpu.VMEM((_N_ORI, _DOCS_PER_W), jnp.float32),
        pltpu.VMEM((_N_ORI, _DOCS_PER_W), jnp.float32),
        pltpu.VMEM((_HWORDS,), jnp.float32),
        pltpu.VMEM((_DOCS_PER_W,), jnp.float32),
        pltpu.VMEM((_DOCS_PER_W,), jnp.float32),
        pltpu.SemaphoreType.DMA,
        pltpu.SemaphoreType.DMA,
    ],
)(_sc_body)


def kernel(images, W_mlp, b_mlp, W_final, b_final):
    planes = jnp.transpose(images, (1, 3, 2, 0))
    wtab = jnp.concatenate(
        [W_mlp[:, 0], jnp.zeros((_TAB - _N_DEPTH * _N_BINS,), jnp.float32)])
    wsum = W_final[0] + W_final[1] + W_final[2]
    a = wsum / jnp.float32(_N_GEN)
    cc = b_mlp[0] * wsum + b_final
    params = jnp.concatenate([
        jnp.full((16,), a[0], jnp.float32),
        jnp.full((16,), a[1], jnp.float32),
        jnp.full((16,), cc[0], jnp.float32),
        jnp.full((16,), cc[1], jnp.float32),
    ])
    out0, out1 = _sc_call(planes, wtab, params)
    return jnp.stack([out0, out1], axis=-1)


# speedup vs baseline: 1.3697x; 1.3697x over previous
import functools

import jax
import jax.numpy as jnp
from jax import lax
from jax.experimental import pallas as pl
from jax.experimental.pallas import tpu as pltpu
from jax.experimental.pallas import tpu_sc as plsc

_N = 4096
_N_DEPTH = 3
_N_ORI = 100
_N_GEN = 10
_N_BINS = 50
_NPLANE = _N_DEPTH * _N_GEN
_NW = 32
_DOCS_PER_W = _N // _NW
_NQ = _DOCS_PER_W // 16
_TAB = 160


def _splat(ref, i):
    return plsc.load_gather(ref, [jnp.full((16,), i, jnp.int32)])


def _sc_body(planes_h, wm_h, bm_h, wf_h, bf_h, out0_h, out1_h,
             wtab, small, buf0, buf1, row0, row1, sem0, sem1):
    c = lax.axis_index("c")
    s = lax.axis_index("s")
    wid = s * 2 + c
    col = wid * _DOCS_PER_W

    pltpu.sync_copy(wm_h, wtab.at[pl.ds(0, _N_DEPTH * _N_BINS)])
    pltpu.sync_copy(wf_h, small.at[pl.ds(0, 6)])
    pltpu.sync_copy(bf_h, small.at[pl.ds(8, 2)])
    pltpu.sync_copy(bm_h, small.at[pl.ds(16, 1)])

    bufs = (buf0, buf1)
    sems = (sem0, sem1)
    handles = [None, None]
    handles[0] = pltpu.async_copy(
        planes_h.at[0, 0, slice(None), pl.ds(col, _DOCS_PER_W)],
        bufs[0], sems[0])

    accs = tuple(jnp.zeros((16,), jnp.float32) for _ in range(_NQ))

    for p in range(_NPLANE):
        b = p & 1
        handles[b].wait()
        if p + 1 < _NPLANE:
            d1, g1 = divmod(p + 1, _N_GEN)
            handles[1 - b] = pltpu.async_copy(
                planes_h.at[d1, g1, slice(None), pl.ds(col, _DOCS_PER_W)],
                bufs[1 - b], sems[1 - b])

        buf = bufs[b]
        off = (p // _N_GEN) * _N_BINS

        @plsc.parallel_loop(0, _N_ORI, unroll=2, carry=accs)
        def body(r, acc_t, _buf=buf, _off=off):
            new = []
            for q in range(_NQ):
                x = _buf[r, pl.ds(16 * q, 16)]
                t = jnp.minimum(x * jnp.float32(_N_BINS),
                                jnp.float32(_N_BINS - 1))
                ti = t.astype(jnp.int32)
                if _off:
                    ti = ti + _off
                w = plsc.load_gather(wtab, [ti])
                new.append(acc_t[q] + w)
            return tuple(new)

        accs = body

    ws0 = _splat(small, 0) + _splat(small, 2) + _splat(small, 4)
    ws1 = _splat(small, 1) + _splat(small, 3) + _splat(small, 5)
    bm = _splat(small, 16)
    a0 = ws0 * jnp.float32(1.0 / _N_GEN)
    a1 = ws1 * jnp.float32(1.0 / _N_GEN)
    c0 = bm * ws0 + _splat(small, 8)
    c1 = bm * ws1 + _splat(small, 9)
    for q in range(_NQ):
        row0[pl.ds(q * 16, 16)] = accs[q] * a0 + c0
        row1[pl.ds(q * 16, 16)] = accs[q] * a1 + c1

    pltpu.sync_copy(row0, out0_h.at[pl.ds(col, _DOCS_PER_W)])
    pltpu.sync_copy(row1, out1_h.at[pl.ds(col, _DOCS_PER_W)])


_mesh = plsc.VectorSubcoreMesh(core_axis_name="c", subcore_axis_name="s")

_sc_call = functools.partial(
    pl.kernel,
    mesh=_mesh,
    compiler_params=pltpu.CompilerParams(needs_layout_passes=False),
    out_type=[
        jax.ShapeDtypeStruct((_N,), jnp.float32),
        jax.ShapeDtypeStruct((_N,), jnp.float32),
    ],
    scratch_types=[
        pltpu.VMEM((_TAB,), jnp.float32),
        pltpu.VMEM((32,), jnp.float32),
        pltpu.VMEM((_N_ORI, _DOCS_PER_W), jnp.float32),
        pltpu.VMEM((_N_ORI, _DOCS_PER_W), jnp.float32),
        pltpu.VMEM((_DOCS_PER_W,), jnp.float32),
        pltpu.VMEM((_DOCS_PER_W,), jnp.float32),
        pltpu.SemaphoreType.DMA,
        pltpu.SemaphoreType.DMA,
    ],
)(_sc_body)


def kernel(images, W_mlp, b_mlp, W_final, b_final):
    planes = jnp.transpose(images, (1, 3, 2, 0))
    out0, out1 = _sc_call(planes, W_mlp.reshape(_N_DEPTH * _N_BINS),
                          b_mlp, W_final.reshape(6), b_final)
    return jnp.stack([out0, out1], axis=-1)


# speedup vs baseline: 1.3771x; 1.0054x over previous
import functools

import jax
import jax.numpy as jnp
from jax import lax
from jax.experimental import pallas as pl
from jax.experimental.pallas import tpu as pltpu
from jax.experimental.pallas import tpu_sc as plsc

_N = 4096
_N_DEPTH = 3
_N_ORI = 100
_N_GEN = 10
_N_BINS = 50
_NPLANE = _N_DEPTH * _N_GEN
_NW = 32
_DOCS_PER_W = _N // _NW
_NQ = _DOCS_PER_W // 16
_TAB = 160


def _splat(ref, i):
    return plsc.load_gather(ref, [jnp.full((16,), i, jnp.int32)])


def _sc_body(planes_h, wm_h, wf_h, out0_h, out1_h,
             wtab, small, buf0, buf1, row0, row1, sem0, sem1):
    c = lax.axis_index("c")
    s = lax.axis_index("s")
    wid = s * 2 + c
    col = wid * _DOCS_PER_W

    pltpu.sync_copy(wm_h, wtab)
    pltpu.sync_copy(wf_h, small)

    bufs = (buf0, buf1)
    sems = (sem0, sem1)
    handles = [None, None]
    handles[0] = pltpu.async_copy(
        planes_h.at[0, 0, slice(None), pl.ds(col, _DOCS_PER_W)],
        bufs[0], sems[0])

    accs = tuple(jnp.zeros((16,), jnp.float32) for _ in range(_NQ))

    for p in range(_NPLANE):
        b = p & 1
        handles[b].wait()
        if p + 1 < _NPLANE:
            d1, g1 = divmod(p + 1, _N_GEN)
            handles[1 - b] = pltpu.async_copy(
                planes_h.at[d1, g1, slice(None), pl.ds(col, _DOCS_PER_W)],
                bufs[1 - b], sems[1 - b])

        buf = bufs[b]
        off = (p // _N_GEN) * _N_BINS

        @plsc.parallel_loop(0, _N_ORI, unroll=2, carry=accs)
        def body(r, acc_t, _buf=buf, _off=off):
            new = []
            for q in range(_NQ):
                x = _buf[r, pl.ds(16 * q, 16)]
                t = jnp.minimum(x * jnp.float32(_N_BINS),
                                jnp.float32(_N_BINS - 1))
                ti = t.astype(jnp.int32)
                if _off:
                    ti = ti + _off
                w = plsc.load_gather(wtab, [ti])
                new.append(acc_t[q] + w)
            return tuple(new)

        accs = body

    ws0 = _splat(small, 0) + _splat(small, 2) + _splat(small, 4)
    ws1 = _splat(small, 1) + _splat(small, 3) + _splat(small, 5)
    bm = _splat(small, 8)
    a0 = ws0 * jnp.float32(1.0 / _N_GEN)
    a1 = ws1 * jnp.float32(1.0 / _N_GEN)
    c0 = bm * ws0 + _splat(small, 6)
    c1 = bm * ws1 + _splat(small, 7)
    for q in range(_NQ):
        row0[pl.ds(q * 16, 16)] = accs[q] * a0 + c0
        row1[pl.ds(q * 16, 16)] = accs[q] * a1 + c1

    pltpu.sync_copy(row0, out0_h.at[pl.ds(col, _DOCS_PER_W)])
    pltpu.sync_copy(row1, out1_h.at[pl.ds(col, _DOCS_PER_W)])


_mesh = plsc.VectorSubcoreMesh(core_axis_name="c", subcore_axis_name="s")

_sc_call = functools.partial(
    pl.kernel,
    mesh=_mesh,
    compiler_params=pltpu.CompilerParams(needs_layout_passes=False),
    out_type=[
        jax.ShapeDtypeStruct((_N,), jnp.float32),
        jax.ShapeDtypeStruct((_N,), jnp.float32),
    ],
    scratch_types=[
        pltpu.VMEM((_TAB,), jnp.float32),
        pltpu.VMEM((16,), jnp.float32),
        pltpu.VMEM((_N_ORI, _DOCS_PER_W), jnp.float32),
        pltpu.VMEM((_N_ORI, _DOCS_PER_W), jnp.float32),
        pltpu.VMEM((_DOCS_PER_W,), jnp.float32),
        pltpu.VMEM((_DOCS_PER_W,), jnp.float32),
        pltpu.SemaphoreType.DMA,
        pltpu.SemaphoreType.DMA,
    ],
)(_sc_body)


def kernel(images, W_mlp, b_mlp, W_final, b_final):
    planes = jnp.transpose(images, (1, 3, 2, 0))
    wtab = jnp.concatenate(
        [W_mlp[:, 0], jnp.zeros((_TAB - _N_DEPTH * _N_BINS,), jnp.float32)])
    small = jnp.concatenate(
        [W_final.reshape(6), b_final, b_mlp,
         jnp.zeros((7,), jnp.float32)])
    out0, out1 = _sc_call(planes, wtab, small)
    return jnp.stack([out0, out1], axis=-1)
